# Initial kernel scaffold; baseline (speedup 1.0000x reference)
import jax
import jax.numpy as jnp
from jax.experimental import pallas as pl


def kernel(inputs, embed_table):
    raise NotImplementedError("write your pallas kernel here")


# speedup vs baseline: 2.5972x; 2.5972x over previous
import numpy as np
import jax
import jax.numpy as jnp
from jax import lax
from jax.experimental import pallas as pl
from jax.experimental.pallas import tpu as pltpu
from jax.experimental.pallas import tpu_sc as plsc

MAX_SEQ_LEN = 200
EMBED_DIM = 64
BATCH = 4096
_LANES = 16

_NC, _NS = 2, 16
_NW = _NC * _NS
_N = BATCH * MAX_SEQ_LEN
_S = 128
_CHUNK = 2 * _S
_NCHUNK = _N // _CHUNK
_CPW = _NCHUNK // _NW
_POS3 = 3 * MAX_SEQ_LEN


def _pos_encoding(max_seq_len, wordvec_size):
    pos = np.arange(max_seq_len).reshape(1, -1).T
    i = np.arange(wordvec_size / 2).reshape(1, -1)
    pos_emb = np.empty((max_seq_len, wordvec_size))
    pos_emb[:, 0::2] = np.sin(pos / np.power(10000, 2 * i / wordvec_size))
    pos_emb[:, 1::2] = np.cos(pos / np.power(10000, 2 * i / wordvec_size))
    return pos_emb.astype(np.float32)


_POS = _pos_encoding(MAX_SEQ_LEN, EMBED_DIM)


def _sc_gather_posadd(table, idx2d, pos3):
    mesh = plsc.VectorSubcoreMesh(core_axis_name="c", subcore_axis_name="s")

    @pl.kernel(
        out_type=jax.ShapeDtypeStruct((_N, EMBED_DIM), jnp.float32),
        mesh=mesh,
        compiler_params=pltpu.CompilerParams(use_tc_tiling_on_sc=False),
        scratch_types=[
            pltpu.VMEM((2, 2, _S), jnp.int32),
            pltpu.VMEM((2, _CHUNK, EMBED_DIM), jnp.float32),
            pltpu.VMEM((_POS3, EMBED_DIM), jnp.float32),
            pltpu.SemaphoreType.DMA,
            pltpu.SemaphoreType.DMA,
            pltpu.SemaphoreType.DMA,
            pltpu.SemaphoreType.DMA,
            pltpu.SemaphoreType.DMA,
            pltpu.SemaphoreType.DMA,
        ],
    )
    def k(table_hbm, idx_hbm, pos_hbm, out_hbm, idx_v, rows_v, pos_v,
          si0, si1, sg0, sg1, so0, so1):
        wid = lax.axis_index("s") * _NC + lax.axis_index("c")
        base = wid * _CPW
        s_i = (si0, si1)
        s_g = (sg0, sg1)
        s_o = (so0, so1)

        pltpu.sync_copy(pos_hbm, pos_v)

        def idx_copy(c, b):
            return pltpu.make_async_copy(
                idx_hbm.at[pl.ds((base + c) * 2, 2)],
                idx_v.at[b], s_i[b])

        def gathers(b):
            g1 = pltpu.make_async_copy(
                table_hbm.at[idx_v.at[b, 0]],
                rows_v.at[b, pl.ds(0, _S)], s_g[b])
            g2 = pltpu.make_async_copy(
                table_hbm.at[idx_v.at[b, 1]],
                rows_v.at[b, pl.ds(_S, _S)], s_g[b])
            return g1, g2

        def out_copy(c, b):
            return pltpu.make_async_copy(
                rows_v.at[b],
                out_hbm.at[pl.ds((base + c) * _CHUNK, _CHUNK)],
                s_o[b])

        idx_copy(0, 0).start()
        idx_copy(1, 1).start()
        idx_copy(0, 0).wait()
        g1, g2 = gathers(0)
        g1.start()
        g2.start()

        @pl.loop(0, _CPW, step=2)
        def _(cc):
            for b in range(2):
                c = cc + b
                nb = 1 - b

                g1, g2 = gathers(b)
                g1.wait()
                g2.wait()

                @pl.when(c + 2 < _CPW)
                def _():
                    idx_copy(c + 2, b).start()

                p0 = lax.rem((base + c) * _CHUNK, MAX_SEQ_LEN)

                @pl.loop(0, _CHUNK)
                def _(r):
                    for cl in range(EMBED_DIM // _LANES):
                        sl = pl.ds(cl * _LANES, _LANES)
                        plsc.addupdate(rows_v.at[b, r, sl], pos_v[p0 + r, sl])

                out_copy(c, b).start()

                @pl.when(c + 1 < _CPW)
                def _():
                    idx_copy(c + 1, nb).wait()

                    @pl.when(c >= 1)
                    def _():
                        out_copy(c - 1, nb).wait()

                    ng1, ng2 = gathers(nb)
                    ng1.start()
                    ng2.start()

        out_copy(_CPW - 2, 0).wait()
        out_copy(_CPW - 1, 1).wait()

    return k(table, idx2d, pos3)


def _mask_body(x_ref, o_ref):
    o_ref[...] = (x_ref[...] != 0).astype(jnp.float32)


def _padding_mask(inputs):
    return pl.pallas_call(
        _mask_body,
        out_shape=jax.ShapeDtypeStruct((BATCH, MAX_SEQ_LEN), jnp.float32),
    )(inputs)


def kernel(inputs, embed_table):
    idx2d = inputs.reshape(_N // _S, _S)
    pos3 = jnp.tile(jnp.asarray(_POS), (3, 1))
    out = _sc_gather_posadd(embed_table, idx2d, pos3)
    mask = _padding_mask(inputs)
    return (
        out.reshape(BATCH, MAX_SEQ_LEN, EMBED_DIM),
        mask.reshape(BATCH, 1, MAX_SEQ_LEN),
    )


# speedup vs baseline: 3.2392x; 1.2472x over previous
import numpy as np
import jax
import jax.numpy as jnp
from jax import lax
from jax.experimental import pallas as pl
from jax.experimental.pallas import tpu as pltpu
from jax.experimental.pallas import tpu_sc as plsc

MAX_SEQ_LEN = 200
EMBED_DIM = 64
BATCH = 4096
VOCAB = 100000

_N = BATCH * MAX_SEQ_LEN
_S = 128
_NCH = _N // _S
_NC, _NS = 2, 16
_NW = _NC * _NS
_CPW = _NCH // _NW


def _pos_encoding(max_seq_len, wordvec_size):
    pos = np.arange(max_seq_len).reshape(1, -1).T
    i = np.arange(wordvec_size / 2).reshape(1, -1)
    pos_emb = np.empty((max_seq_len, wordvec_size))
    pos_emb[:, 0::2] = np.sin(pos / np.power(10000, 2 * i / wordvec_size))
    pos_emb[:, 1::2] = np.cos(pos / np.power(10000, 2 * i / wordvec_size))
    return pos_emb.astype(np.float32)


_POS = _pos_encoding(MAX_SEQ_LEN, EMBED_DIM)

_TC_PARAMS = pltpu.CompilerParams(dimension_semantics=("parallel",))


def _mask_body(x_ref, o_ref):
    o_ref[...] = (x_ref[...] != 0).astype(jnp.float32)


def _padding_mask(inputs):
    return pl.pallas_call(
        _mask_body,
        out_shape=jax.ShapeDtypeStruct((BATCH, MAX_SEQ_LEN), jnp.float32),
    )(inputs)


def _pack_body(t_ref, o_ref):
    t = t_ref[...]
    o_ref[...] = jnp.concatenate(
        [t, jnp.zeros((t.shape[0], 128 - EMBED_DIM), jnp.float32)], axis=1)


def _tc_pack_table(table):
    blk = 5000
    return pl.pallas_call(
        _pack_body,
        grid=(VOCAB // blk,),
        in_specs=[pl.BlockSpec((blk, EMBED_DIM), lambda i: (i, 0))],
        out_specs=pl.BlockSpec((blk, 128), lambda i: (i, 0)),
        out_shape=jax.ShapeDtypeStruct((VOCAB, 128), jnp.float32),
        compiler_params=_TC_PARAMS,
    )(table)


def _finish_body(g_ref, p_ref, o_ref):
    nb = o_ref.shape[0]
    x = g_ref[...].reshape(nb, MAX_SEQ_LEN, 128)
    o_ref[...] = x[:, :, 0:EMBED_DIM] + p_ref[...][None]


def _tc_finish(gathered, pos):
    blk = 64
    return pl.pallas_call(
        _finish_body,
        grid=(BATCH // blk,),
        in_specs=[
            pl.BlockSpec((blk * MAX_SEQ_LEN, 128), lambda i: (i, 0)),
            pl.BlockSpec((MAX_SEQ_LEN, EMBED_DIM), lambda i: (0, 0)),
        ],
        out_specs=pl.BlockSpec((blk, MAX_SEQ_LEN, EMBED_DIM),
                               lambda i: (i, 0, 0)),
        out_shape=jax.ShapeDtypeStruct((BATCH, MAX_SEQ_LEN, EMBED_DIM),
                                       jnp.float32),
        compiler_params=_TC_PARAMS,
    )(gathered, pos)


def _sc_gather(table128, idx128):
    mesh = plsc.VectorSubcoreMesh(core_axis_name="c", subcore_axis_name="s")

    @pl.kernel(
        out_type=jax.ShapeDtypeStruct((_N, 128), jnp.float32),
        mesh=mesh,
        compiler_params=pltpu.CompilerParams(use_tc_tiling_on_sc=True),
        scratch_types=[
            pltpu.VMEM((_CPW, _S), jnp.int32),
            pltpu.VMEM((4, _S, 128), jnp.float32),
            pltpu.SemaphoreType.DMA,
            pltpu.SemaphoreType.DMA,
            pltpu.SemaphoreType.DMA,
            pltpu.SemaphoreType.DMA,
            pltpu.SemaphoreType.DMA,
            pltpu.SemaphoreType.DMA,
            pltpu.SemaphoreType.DMA,
            pltpu.SemaphoreType.DMA,
        ],
    )
    def k(table_hbm, idx_hbm, out_hbm, idx_v, rows_v,
          sg0, sg1, sg2, sg3, so0, so1, so2, so3):
        wid = lax.axis_index("s") * _NC + lax.axis_index("c")
        base = wid * _CPW
        s_g = (sg0, sg1, sg2, sg3)
        s_o = (so0, so1, so2, so3)

        pltpu.sync_copy(idx_hbm.at[pl.ds(base, _CPW)], idx_v)

        def gather(c, b):
            return pltpu.make_async_copy(
                table_hbm.at[idx_v.at[c]], rows_v.at[b], s_g[b])

        def out_copy(c, b):
            return pltpu.make_async_copy(
                rows_v.at[b],
                out_hbm.at[pl.ds((base + c) * _S, _S)], s_o[b])

        gather(0, 0).start()
        gather(1, 1).start()

        @pl.loop(0, _CPW, step=4)
        def _(cc):
            for j in range(4):
                c = cc + j
                b = j
                nb = (j + 2) % 4

                gather(c, b).wait()
                out_copy(c, b).start()

                @pl.when(c >= 2)
                def _():
                    out_copy(c - 2, nb).wait()

                @pl.when(c + 2 < _CPW)
                def _():
                    gather(c + 2, nb).start()

        out_copy(_CPW - 2, 2).wait()
        out_copy(_CPW - 1, 3).wait()

    return k(table128, idx128)


def kernel(inputs, embed_table):
    mask = _padding_mask(inputs)
    table128 = _tc_pack_table(embed_table)
    idx128 = inputs.reshape(_NCH, _S)
    gathered = _sc_gather(table128, idx128)
    out = _tc_finish(gathered, jnp.asarray(_POS))
    return (out, mask.reshape(BATCH, 1, MAX_SEQ_LEN))
